# Initial kernel scaffold; baseline (speedup 1.0000x reference)
import jax
import jax.numpy as jnp
from jax.experimental import pallas as pl


def kernel(x, depth, type_table, attr_table):
    raise NotImplementedError("write your pallas kernel here")


# speedup vs baseline: 2.2307x; 2.2307x over previous
import jax
import jax.numpy as jnp
from jax import lax
from jax.experimental import pallas as pl
from jax.experimental.pallas import tpu as pltpu
from jax.experimental.pallas import tpu_sc as plsc

_N = 100000
_D = 128
_C = 128
_NW = 32
_FULL = _N // _C
_BASE_CHUNKS = _FULL // _NW
_EXTRA = _FULL - _BASE_CHUNKS * _NW
_TAIL = _N - _FULL * _C
_TAIL_BASE = _FULL * _C
_L = 16


def _sc_body(t_hbm, a_hbm, type_hbm, attr_hbm, out_hbm,
             idx_t, idx_a, buf_t, buf_a, sem_t, sem_a):
    wid = lax.axis_index("s") * 2 + lax.axis_index("c")

    def do_rows(base, rows):
        i_t = idx_t if rows == _C else idx_t.at[pl.ds(0, rows)]
        i_a = idx_a if rows == _C else idx_a.at[pl.ds(0, rows)]
        b_t = buf_t if rows == _C else buf_t.at[pl.ds(0, rows)]
        b_a = buf_a if rows == _C else buf_a.at[pl.ds(0, rows)]
        pltpu.sync_copy(t_hbm.at[pl.ds(base, rows)], i_t)
        pltpu.sync_copy(a_hbm.at[pl.ds(base, rows)], i_a)
        ct = pltpu.async_copy(type_hbm.at[i_t], b_t, sem_t)
        ca = pltpu.async_copy(attr_hbm.at[i_a], b_a, sem_a)
        ct.wait()
        ca.wait()

        @pl.loop(0, rows)
        def _(r):
            for c in range(_D // _L):
                sl = (pl.ds(r, 1), pl.ds(c * _L, _L))
                buf_a.at[sl][...] = buf_a.at[sl][...] + buf_t.at[sl][...]

        pltpu.sync_copy(b_a, out_hbm.at[pl.ds(base, rows)])

    @pl.loop(0, _BASE_CHUNKS)
    def _(i):
        chunk = wid + i * _NW
        do_rows(pl.multiple_of(chunk * _C, _C), _C)

    @pl.when(wid < _EXTRA)
    def _():
        do_rows(pl.multiple_of((_BASE_CHUNKS * _NW + wid) * _C, _C), _C)

    @pl.when(wid == _NW - 1)
    def _():
        do_rows(_TAIL_BASE, _TAIL)


def kernel(x, depth, type_table, attr_table):
    del depth
    t_idx = x[:, 0].astype(jnp.int32)
    a_idx = x[:, 1].astype(jnp.int32)
    mesh = plsc.VectorSubcoreMesh(core_axis_name="c", subcore_axis_name="s")
    run = pl.kernel(
        _sc_body,
        out_type=jax.ShapeDtypeStruct((_N, _D), jnp.float32),
        mesh=mesh,
        scratch_types=[
            pltpu.VMEM((_C,), jnp.int32),
            pltpu.VMEM((_C,), jnp.int32),
            pltpu.VMEM((_C, _D), jnp.float32),
            pltpu.VMEM((_C, _D), jnp.float32),
            pltpu.SemaphoreType.DMA,
            pltpu.SemaphoreType.DMA,
        ],
    )
    return run(t_idx, a_idx, type_table, attr_table)
